# Initial kernel scaffold; baseline (speedup 1.0000x reference)
import jax
import jax.numpy as jnp
from jax.experimental import pallas as pl


def kernel(z, centroids):
    raise NotImplementedError("write your pallas kernel here")


# speedup vs baseline: 1.4038x; 1.4038x over previous
import jax
import jax.numpy as jnp
from jax.experimental import pallas as pl
from jax.experimental.pallas import tpu as pltpu

_EPS = 1e-12
_BN = 512
_BN2 = 1024


def _c2_kernel(ct_ref, c2_ref):
    c = ct_ref[...]
    c2_ref[...] = jnp.sum(c * c, axis=0, keepdims=True)


def _q_kernel(z_ref, ct_ref, c2_ref, q_ref, fpart_ref):
    z = z_ref[...]
    z2 = jnp.sum(z * z, axis=1, keepdims=True)
    zb = z.astype(jnp.bfloat16)
    dot = jnp.dot(zb, ct_ref[...], preferred_element_type=jnp.float32)
    d2 = (z2 + c2_ref[...]) - 2.0 * dot
    dist = jnp.sqrt(jnp.maximum(d2, _EPS))
    num = 1.0 / (1.0 + dist)
    s = jnp.sum(num, axis=1, keepdims=True)
    q = num * (1.0 / s)
    q_ref[...] = q
    fpart_ref[...] = jnp.sum(q, axis=0, keepdims=True)[None]


def _p_kernel(q_ref, fpart_ref, p_ref):
    f = jnp.sum(fpart_ref[...], axis=(0, 1), keepdims=False)
    q = q_ref[...]
    u = (q * q) * (1.0 / f)[None, :]
    s = jnp.sum(u, axis=1, keepdims=True)
    p_ref[...] = u * (1.0 / s)


def kernel(z, centroids):
    n, d = z.shape
    k = centroids.shape[0]
    nb = n // _BN

    ct32 = centroids.T
    ct16 = ct32.astype(jnp.bfloat16)

    c2 = pl.pallas_call(
        _c2_kernel,
        out_shape=jax.ShapeDtypeStruct((1, k), jnp.float32),
        name="c2",
    )(ct32)

    q, fpart = pl.pallas_call(
        _q_kernel,
        grid=(nb,),
        in_specs=[
            pl.BlockSpec((_BN, d), lambda i: (i, 0)),
            pl.BlockSpec((d, k), lambda i: (0, 0)),
            pl.BlockSpec((1, k), lambda i: (0, 0)),
        ],
        out_specs=[
            pl.BlockSpec((_BN, k), lambda i: (i, 0)),
            pl.BlockSpec((1, 1, k), lambda i: (i, 0, 0)),
        ],
        out_shape=[
            jax.ShapeDtypeStruct((n, k), jnp.float32),
            jax.ShapeDtypeStruct((nb, 1, k), jnp.float32),
        ],
        compiler_params=pltpu.CompilerParams(
            dimension_semantics=("parallel",),
        ),
        name="q_assign",
    )(z, ct16, c2)

    p = pl.pallas_call(
        _p_kernel,
        grid=(n // _BN2,),
        in_specs=[
            pl.BlockSpec((_BN2, k), lambda i: (i, 0)),
            pl.BlockSpec((nb, 1, k), lambda i: (0, 0, 0)),
        ],
        out_specs=pl.BlockSpec((_BN2, k), lambda i: (i, 0)),
        out_shape=jax.ShapeDtypeStruct((n, k), jnp.float32),
        compiler_params=pltpu.CompilerParams(
            dimension_semantics=("parallel",),
        ),
        name="p_target",
    )(q, fpart)

    return (z, q, p)


# speedup vs baseline: 1.9198x; 1.3676x over previous
import jax
import jax.numpy as jnp
from jax.experimental import pallas as pl
from jax.experimental.pallas import tpu as pltpu

_EPS = 1e-12
_BN = 512
_BN2 = 1024


def _c2_kernel(ct_ref, c2_ref):
    c = ct_ref[...]
    c2_ref[...] = jnp.sum(c * c, axis=0, keepdims=True)


def _q_kernel(z_ref, ct_ref, c2_ref, q_ref, fpart_ref, zout_ref):
    z = z_ref[...]
    zout_ref[...] = z
    z2 = jnp.sum(z * z, axis=1, keepdims=True)
    zb = z.astype(jnp.bfloat16)
    dot = jnp.dot(zb, ct_ref[...], preferred_element_type=jnp.float32)
    d2 = (z2 + c2_ref[...]) - 2.0 * dot
    dist = jnp.sqrt(jnp.maximum(d2, _EPS))
    num = 1.0 / (1.0 + dist)
    s = jnp.sum(num, axis=1, keepdims=True)
    q = num * (1.0 / s)
    q_ref[...] = q
    fpart_ref[...] = jnp.sum(q, axis=0, keepdims=True)[None]


def _p_kernel(q_ref, fpart_ref, p_ref):
    f = jnp.sum(fpart_ref[...], axis=(0, 1), keepdims=False)
    q = q_ref[...]
    u = (q * q) * (1.0 / f)[None, :]
    s = jnp.sum(u, axis=1, keepdims=True)
    p_ref[...] = u * (1.0 / s)


def kernel(z, centroids):
    n, d = z.shape
    k = centroids.shape[0]
    nb = n // _BN

    ct32 = centroids.T
    ct16 = ct32.astype(jnp.bfloat16)

    c2 = pl.pallas_call(
        _c2_kernel,
        out_shape=jax.ShapeDtypeStruct((1, k), jnp.float32),
        name="c2",
    )(ct32)

    q, fpart, z_out = pl.pallas_call(
        _q_kernel,
        grid=(nb,),
        in_specs=[
            pl.BlockSpec((_BN, d), lambda i: (i, 0)),
            pl.BlockSpec((d, k), lambda i: (0, 0)),
            pl.BlockSpec((1, k), lambda i: (0, 0)),
        ],
        out_specs=[
            pl.BlockSpec((_BN, k), lambda i: (i, 0)),
            pl.BlockSpec((1, 1, k), lambda i: (i, 0, 0)),
            pl.BlockSpec((_BN, d), lambda i: (i, 0)),
        ],
        out_shape=[
            jax.ShapeDtypeStruct((n, k), jnp.float32),
            jax.ShapeDtypeStruct((nb, 1, k), jnp.float32),
            jax.ShapeDtypeStruct((n, d), jnp.float32),
        ],
        compiler_params=pltpu.CompilerParams(
            dimension_semantics=("parallel",),
        ),
        name="q_assign",
    )(z, ct16, c2)

    p = pl.pallas_call(
        _p_kernel,
        grid=(n // _BN2,),
        in_specs=[
            pl.BlockSpec((_BN2, k), lambda i: (i, 0)),
            pl.BlockSpec((nb, 1, k), lambda i: (0, 0, 0)),
        ],
        out_specs=pl.BlockSpec((_BN2, k), lambda i: (i, 0)),
        out_shape=jax.ShapeDtypeStruct((n, k), jnp.float32),
        compiler_params=pltpu.CompilerParams(
            dimension_semantics=("parallel",),
        ),
        name="p_target",
    )(q, fpart)

    return (z_out, q, p)


# speedup vs baseline: 2.0320x; 1.0584x over previous
import jax
import jax.numpy as jnp
from jax.experimental import pallas as pl
from jax.experimental.pallas import tpu as pltpu

_EPS = 1e-12
_BN = 512
_BN2 = 2048


def _c2_kernel(ct_ref, c2_ref):
    c = ct_ref[...]
    c2_ref[...] = jnp.sum(c * c, axis=0, keepdims=True)


def _q_kernel(z_ref, ct_ref, ones_ref, c2_ref, q_ref, fpart_ref, zout_ref):
    z = z_ref[...]
    zout_ref[...] = z
    zb = z.astype(jnp.bfloat16)
    z2 = jnp.dot(zb * zb, ones_ref[...], preferred_element_type=jnp.float32)
    dot = jnp.dot(zb, ct_ref[...], preferred_element_type=jnp.float32)
    d2 = (z2 + c2_ref[...]) - 2.0 * dot
    d2c = jnp.maximum(d2, _EPS)
    dist = d2c * jax.lax.rsqrt(d2c)
    num = 1.0 / (1.0 + dist)
    s = jnp.sum(num, axis=1, keepdims=True)
    q = num * (1.0 / s)
    q_ref[...] = q
    fpart_ref[...] = jnp.sum(q, axis=0, keepdims=True)[None]


def _p_kernel(q_ref, fpart_ref, p_ref):
    f = jnp.sum(fpart_ref[...], axis=(0, 1), keepdims=False)
    q = q_ref[...]
    u = (q * q) * (1.0 / f)[None, :]
    s = jnp.sum(u, axis=1, keepdims=True)
    p_ref[...] = u * (1.0 / s)


def kernel(z, centroids):
    n, d = z.shape
    k = centroids.shape[0]
    nb = n // _BN

    ct32 = centroids.T
    ct16 = ct32.astype(jnp.bfloat16)
    ones16 = jnp.ones((d, k), jnp.bfloat16)

    c2 = pl.pallas_call(
        _c2_kernel,
        out_shape=jax.ShapeDtypeStruct((1, k), jnp.float32),
        name="c2",
    )(ct32)

    q, fpart, z_out = pl.pallas_call(
        _q_kernel,
        grid=(nb,),
        in_specs=[
            pl.BlockSpec((_BN, d), lambda i: (i, 0)),
            pl.BlockSpec((d, k), lambda i: (0, 0)),
            pl.BlockSpec((d, k), lambda i: (0, 0)),
            pl.BlockSpec((1, k), lambda i: (0, 0)),
        ],
        out_specs=[
            pl.BlockSpec((_BN, k), lambda i: (i, 0)),
            pl.BlockSpec((1, 1, k), lambda i: (i, 0, 0)),
            pl.BlockSpec((_BN, d), lambda i: (i, 0)),
        ],
        out_shape=[
            jax.ShapeDtypeStruct((n, k), jnp.float32),
            jax.ShapeDtypeStruct((nb, 1, k), jnp.float32),
            jax.ShapeDtypeStruct((n, d), jnp.float32),
        ],
        compiler_params=pltpu.CompilerParams(
            dimension_semantics=("parallel",),
        ),
        name="q_assign",
    )(z, ct16, ones16, c2)

    p = pl.pallas_call(
        _p_kernel,
        grid=(n // _BN2,),
        in_specs=[
            pl.BlockSpec((_BN2, k), lambda i: (i, 0)),
            pl.BlockSpec((nb, 1, k), lambda i: (0, 0, 0)),
        ],
        out_specs=pl.BlockSpec((_BN2, k), lambda i: (i, 0)),
        out_shape=jax.ShapeDtypeStruct((n, k), jnp.float32),
        compiler_params=pltpu.CompilerParams(
            dimension_semantics=("parallel",),
        ),
        name="p_target",
    )(q, fpart)

    return (z_out, q, p)


# speedup vs baseline: 2.3792x; 1.1708x over previous
import jax
import jax.numpy as jnp
from jax.experimental import pallas as pl
from jax.experimental.pallas import tpu as pltpu

_EPS = 1e-12
_BN = 1024
_BN2 = 4096


def _c2_kernel(ct_ref, c2_ref):
    c = ct_ref[...]
    c2_ref[...] = jnp.sum(c * c, axis=0, keepdims=True)


def _q_kernel(z_ref, ct_ref, ones_ref, c2_ref, q_ref, fpart_ref, zout_ref):
    z = z_ref[...]
    zout_ref[...] = z
    zb = z.astype(jnp.bfloat16)
    z2 = jnp.dot(zb * zb, ones_ref[...], preferred_element_type=jnp.float32)
    dot = jnp.dot(zb, ct_ref[...], preferred_element_type=jnp.float32)
    d2 = (z2 + c2_ref[...]) - 2.0 * dot
    d2c = jnp.maximum(d2, _EPS)
    dist = d2c * jax.lax.rsqrt(d2c)
    num = 1.0 / (1.0 + dist)
    s = jnp.sum(num, axis=1, keepdims=True)
    q = num * (1.0 / s)
    q_ref[...] = q
    fpart_ref[...] = jnp.sum(q, axis=0, keepdims=True)[None]


def _p_kernel(q_ref, fpart_ref, p_ref):
    f = jnp.sum(fpart_ref[...], axis=(0, 1), keepdims=False)
    q = q_ref[...]
    u = (q * q) * (1.0 / f)[None, :]
    s = jnp.sum(u, axis=1, keepdims=True)
    p_ref[...] = u * (1.0 / s)


def kernel(z, centroids):
    n, d = z.shape
    k = centroids.shape[0]
    nb = n // _BN

    ct32 = centroids.T
    ct16 = ct32.astype(jnp.bfloat16)
    ones16 = jnp.ones((d, k), jnp.bfloat16)

    c2 = pl.pallas_call(
        _c2_kernel,
        out_shape=jax.ShapeDtypeStruct((1, k), jnp.float32),
        name="c2",
    )(ct32)

    q, fpart, z_out = pl.pallas_call(
        _q_kernel,
        grid=(nb,),
        in_specs=[
            pl.BlockSpec((_BN, d), lambda i: (i, 0)),
            pl.BlockSpec((d, k), lambda i: (0, 0)),
            pl.BlockSpec((d, k), lambda i: (0, 0)),
            pl.BlockSpec((1, k), lambda i: (0, 0)),
        ],
        out_specs=[
            pl.BlockSpec((_BN, k), lambda i: (i, 0)),
            pl.BlockSpec((1, 1, k), lambda i: (i, 0, 0)),
            pl.BlockSpec((_BN, d), lambda i: (i, 0)),
        ],
        out_shape=[
            jax.ShapeDtypeStruct((n, k), jnp.float32),
            jax.ShapeDtypeStruct((nb, 1, k), jnp.float32),
            jax.ShapeDtypeStruct((n, d), jnp.float32),
        ],
        compiler_params=pltpu.CompilerParams(
            dimension_semantics=("parallel",),
        ),
        name="q_assign",
    )(z, ct16, ones16, c2)

    p = pl.pallas_call(
        _p_kernel,
        grid=(n // _BN2,),
        in_specs=[
            pl.BlockSpec((_BN2, k), lambda i: (i, 0)),
            pl.BlockSpec((nb, 1, k), lambda i: (0, 0, 0)),
        ],
        out_specs=pl.BlockSpec((_BN2, k), lambda i: (i, 0)),
        out_shape=jax.ShapeDtypeStruct((n, k), jnp.float32),
        compiler_params=pltpu.CompilerParams(
            dimension_semantics=("parallel",),
        ),
        name="p_target",
    )(q, fpart)

    return (z_out, q, p)


# speedup vs baseline: 2.4430x; 1.0269x over previous
import jax
import jax.numpy as jnp
from jax.experimental import pallas as pl
from jax.experimental.pallas import tpu as pltpu

_EPS = 1e-12
_BN = 2048
_BN2 = 8192


def _c2_kernel(ct_ref, c2_ref):
    c = ct_ref[...]
    c2_ref[...] = jnp.sum(c * c, axis=0, keepdims=True)


def _q_kernel(z_ref, ct_ref, ones_ref, c2_ref, q_ref, fpart_ref, zout_ref):
    z = z_ref[...]
    zout_ref[...] = z
    zb = z.astype(jnp.bfloat16)
    z2 = jnp.dot(zb * zb, ones_ref[...], preferred_element_type=jnp.float32)
    dot = jnp.dot(zb, ct_ref[...], preferred_element_type=jnp.float32)
    d2 = (z2 + c2_ref[...]) - 2.0 * dot
    d2c = jnp.maximum(d2, _EPS)
    dist = d2c * jax.lax.rsqrt(d2c)
    num = 1.0 / (1.0 + dist)
    s = jnp.sum(num, axis=1, keepdims=True)
    q = num * (1.0 / s)
    q_ref[...] = q
    fpart_ref[...] = jnp.sum(q, axis=0, keepdims=True)[None]


def _p_kernel(q_ref, fpart_ref, p_ref):
    f = jnp.sum(fpart_ref[...], axis=(0, 1), keepdims=False)
    q = q_ref[...]
    u = (q * q) * (1.0 / f)[None, :]
    s = jnp.sum(u, axis=1, keepdims=True)
    p_ref[...] = u * (1.0 / s)


def kernel(z, centroids):
    n, d = z.shape
    k = centroids.shape[0]
    nb = n // _BN

    ct32 = centroids.T
    ct16 = ct32.astype(jnp.bfloat16)
    ones16 = jnp.ones((d, k), jnp.bfloat16)

    c2 = pl.pallas_call(
        _c2_kernel,
        out_shape=jax.ShapeDtypeStruct((1, k), jnp.float32),
        name="c2",
    )(ct32)

    q, fpart, z_out = pl.pallas_call(
        _q_kernel,
        grid=(nb,),
        in_specs=[
            pl.BlockSpec((_BN, d), lambda i: (i, 0)),
            pl.BlockSpec((d, k), lambda i: (0, 0)),
            pl.BlockSpec((d, k), lambda i: (0, 0)),
            pl.BlockSpec((1, k), lambda i: (0, 0)),
        ],
        out_specs=[
            pl.BlockSpec((_BN, k), lambda i: (i, 0)),
            pl.BlockSpec((1, 1, k), lambda i: (i, 0, 0)),
            pl.BlockSpec((_BN, d), lambda i: (i, 0)),
        ],
        out_shape=[
            jax.ShapeDtypeStruct((n, k), jnp.float32),
            jax.ShapeDtypeStruct((nb, 1, k), jnp.float32),
            jax.ShapeDtypeStruct((n, d), jnp.float32),
        ],
        compiler_params=pltpu.CompilerParams(
            dimension_semantics=("parallel",),
        ),
        name="q_assign",
    )(z, ct16, ones16, c2)

    p = pl.pallas_call(
        _p_kernel,
        grid=(n // _BN2,),
        in_specs=[
            pl.BlockSpec((_BN2, k), lambda i: (i, 0)),
            pl.BlockSpec((nb, 1, k), lambda i: (0, 0, 0)),
        ],
        out_specs=pl.BlockSpec((_BN2, k), lambda i: (i, 0)),
        out_shape=jax.ShapeDtypeStruct((n, k), jnp.float32),
        compiler_params=pltpu.CompilerParams(
            dimension_semantics=("parallel",),
        ),
        name="p_target",
    )(q, fpart)

    return (z_out, q, p)
